# baseline (speedup 1.0000x reference)
import jax
import jax.numpy as jnp
from jax import lax
from jax.experimental import pallas as pl
from jax.experimental.pallas import tpu as pltpu
import jax.experimental.pallas.tpu_sc as plsc

B, E, A = 4096, 8, 1000
L = 16
NC, NS = 2, 16
NW = NC * NS
ROWS_PER_W = B // NW
A_PAD = 1024
UNROLL = 4
NBUF = 2


def _sc_body(a_hbm, er_hbm, o_hbm, a_v, e_v, s_v, in_sem, out_sem):
    wid = lax.axis_index("s") * NC + lax.axis_index("c")
    base = wid * ROWS_PER_W
    lane = lax.iota(jnp.int32, L)

    def start_in(j, row):
        pltpu.async_copy(a_hbm.at[row], a_v.at[j], in_sem.at[j])
        pltpu.async_copy(er_hbm.at[row], e_v.at[j], in_sem.at[j])

    def wait_in(j, row):
        pltpu.make_async_copy(a_hbm.at[row], a_v.at[j], in_sem.at[j]).wait()
        pltpu.make_async_copy(er_hbm.at[row], e_v.at[j], in_sem.at[j]).wait()

    def compute(j, row):
        def p1(off, m):
            off = pl.multiple_of(off, L)
            vals = [a_v[j, e, pl.ds(off, L)] for e in range(E)]
            stds = [e_v[j, e, pl.ds(off, L)] for e in range(E)]
            while len(vals) > 1:
                nv, ns = [], []
                for k in range(0, len(vals), 2):
                    take = vals[k + 1] < vals[k]
                    nv.append(jnp.where(take, vals[k + 1], vals[k]))
                    ns.append(jnp.where(take, stds[k + 1], stds[k]))
                vals, stds = nv, ns
            scaled = vals[0] / jnp.maximum(stds[0], 1e-8)
            scaled = jnp.where(lane + off < A, scaled, -3e38)
            s_v[j, pl.ds(off, L)] = scaled
            return jnp.maximum(m, scaled)

        m = plsc.parallel_loop(
            0, A_PAD, L, unroll=UNROLL,
            carry=jnp.full((L,), -3e38, jnp.float32))(p1)
        row_max = jnp.max(m)

        def p2(off, acc):
            off = pl.multiple_of(off, L)
            v = s_v[j, pl.ds(off, L)]
            p = jnp.exp(v - row_max)
            p = jnp.where(lane + off < A, p, 0.0)
            s_v[j, pl.ds(off, L)] = p
            return acc + p

        acc = plsc.parallel_loop(
            0, A_PAD, L, unroll=UNROLL,
            carry=jnp.zeros((L,), jnp.float32))(p2)
        inv = 1.0 / (jnp.sum(acc) + jnp.zeros((L,), jnp.float32))

        def p3(off):
            off = pl.multiple_of(off, L)
            s_v[j, pl.ds(off, L)] = s_v[j, pl.ds(off, L)] * inv

        plsc.parallel_loop(0, A_PAD, L, unroll=UNROLL)(p3)

    for j in range(NBUF):
        start_in(j, base + j)

    def blk(g, carry):
        for j in range(NBUF):
            i = g * NBUF + j
            row = base + i
            wait_in(j, row)

            @pl.when(g > 0)
            def _():
                pltpu.make_async_copy(
                    s_v.at[j], o_hbm.at[row - NBUF], out_sem.at[j]
                ).wait()

            compute(j, row)
            pltpu.async_copy(s_v.at[j], o_hbm.at[row], out_sem.at[j])

            @pl.when(g < ROWS_PER_W // NBUF - 1)
            def _():
                start_in(j, row + NBUF)
        return carry

    lax.fori_loop(0, ROWS_PER_W // NBUF, blk, 0)

    for j in range(NBUF):
        row = base + ROWS_PER_W - NBUF + j
        pltpu.make_async_copy(
            s_v.at[j], o_hbm.at[row], out_sem.at[j]
        ).wait()


@jax.jit
def _sc_call(action, explore_rate):
    return pl.kernel(
        _sc_body,
        out_type=jax.ShapeDtypeStruct((B, A), jnp.float32),
        mesh=plsc.VectorSubcoreMesh(
            core_axis_name="c", subcore_axis_name="s",
            num_cores=NC, num_subcores=NS,
        ),
        scratch_types=[
            pltpu.VMEM((NBUF, E, A), jnp.float32),
            pltpu.VMEM((NBUF, E, A), jnp.float32),
            pltpu.VMEM((NBUF, A), jnp.float32),
            pltpu.SemaphoreType.DMA((NBUF,)),
            pltpu.SemaphoreType.DMA((NBUF,)),
        ],
        compiler_params=pltpu.CompilerParams(
            use_tc_tiling_on_sc=True, needs_layout_passes=False,
        ),
    )(action, explore_rate)


def kernel(action, explore_rate, step, obs):
    del step, obs
    return _sc_call(action, explore_rate)


# speedup vs baseline: 1.7719x; 1.7719x over previous
import jax
import jax.numpy as jnp
from jax import lax
from jax.experimental import pallas as pl
from jax.experimental.pallas import tpu as pltpu
import jax.experimental.pallas.tpu_sc as plsc

B, E, A = 4096, 8, 1000
L = 16
NC, NS = 2, 16
NW = NC * NS
BW = B // NW
NCH = BW // L
AB = 8
NBLK = A // AB
NBUF = 2
NEG = -3e38


def _sc_body(a_hbm, er_hbm, o_hbm, a_v, e_v, p_v, ms_v, in_sem, out_sem, p2_sem):
    wid = lax.axis_index("s") * NC + lax.axis_index("c")
    bw = pl.multiple_of(wid * BW, BW)

    def start_in(j, blk):
        a0 = pl.multiple_of(blk * AB, AB)
        pltpu.async_copy(a_hbm.at[:, pl.ds(a0, AB), pl.ds(bw, BW)], a_v.at[j], in_sem.at[j])
        pltpu.async_copy(er_hbm.at[:, pl.ds(a0, AB), pl.ds(bw, BW)], e_v.at[j], in_sem.at[j])

    def wait_in(j, blk):
        a0 = pl.multiple_of(blk * AB, AB)
        pltpu.make_async_copy(a_hbm.at[:, pl.ds(a0, AB), pl.ds(bw, BW)], a_v.at[j], in_sem.at[j]).wait()
        pltpu.make_async_copy(er_hbm.at[:, pl.ds(a0, AB), pl.ds(bw, BW)], e_v.at[j], in_sem.at[j]).wait()

    def out_slab(j, blk):
        a0 = pl.multiple_of(blk * AB, AB)
        return pltpu.make_async_copy(
            p_v.at[j], o_hbm.at[pl.ds(a0, AB), pl.ds(bw, BW)], out_sem.at[j])

    for j in range(NBUF):
        start_in(j, j)

    def compute_block(j, blk):
        def pa(a, m):
            mo = []
            for c in range(NCH):
                off = c * L
                vals = [a_v[j, e, a, pl.ds(off, L)] for e in range(E)]
                stds = [e_v[j, e, a, pl.ds(off, L)] for e in range(E)]
                while len(vals) > 1:
                    nv, ns_ = [], []
                    for k in range(0, len(vals), 2):
                        take = vals[k + 1] < vals[k]
                        nv.append(jnp.where(take, vals[k + 1], vals[k]))
                        ns_.append(jnp.where(take, stds[k + 1], stds[k]))
                    vals, stds = nv, ns_
                scaled = vals[0] / jnp.maximum(stds[0], 1e-8)
                p_v[j, a, pl.ds(off, L)] = scaled
                mo.append(jnp.maximum(m[c], scaled))
            return tuple(mo)

        m = lax.fori_loop(0, AB, pa,
                          tuple(jnp.full((L,), NEG, jnp.float32) for _ in range(NCH)))

        def pb(a, s):
            so = []
            for c in range(NCH):
                off = c * L
                p = jnp.exp(p_v[j, a, pl.ds(off, L)] - m[c])
                p_v[j, a, pl.ds(off, L)] = p
                so.append(s[c] + p)
            return tuple(so)

        s = lax.fori_loop(0, AB, pb,
                          tuple(jnp.zeros((L,), jnp.float32) for _ in range(NCH)))

        for c in range(NCH):
            st = pl.multiple_of(blk * (2 * BW) + c * L, L)
            ms_v[pl.ds(st, L)] = m[c]
            ms_v[pl.ds(st + BW, L)] = s[c]
        out_slab(j, blk).start()

    NRING = NBLK // NBUF

    def blk_body(g, carry):
        for j in range(NBUF):
            blk = g * NBUF + j
            wait_in(j, blk)

            @pl.when(g > 0)
            def _():
                out_slab(j, blk - NBUF).wait()

            compute_block(j, blk)

            @pl.when(g < NRING - 1)
            def _():
                start_in(j, blk + NBUF)
        return carry

    lax.fori_loop(0, NRING, blk_body, 0)
    start_in(0, NBLK - 1)
    wait_in(0, NBLK - 1)
    out_slab(0, NBLK - 3).wait()
    compute_block(0, NBLK - 1)
    out_slab(1, NBLK - 2).wait()
    out_slab(0, NBLK - 1).wait()

    def red(blk, m):
        return tuple(
            jnp.maximum(m[c], ms_v[pl.ds(pl.multiple_of(blk * (2 * BW) + c * L, L), L)])
            for c in range(NCH))

    m_fin = lax.fori_loop(0, NBLK, red,
                          tuple(jnp.full((L,), NEG, jnp.float32) for _ in range(NCH)))

    def tot(blk, s):
        so = []
        for c in range(NCH):
            st = pl.multiple_of(blk * (2 * BW) + c * L, L)
            so.append(s[c] + ms_v[pl.ds(st + BW, L)] * jnp.exp(ms_v[pl.ds(st, L)] - m_fin[c]))
        return tuple(so)

    s_tot = lax.fori_loop(0, NBLK, tot,
                          tuple(jnp.zeros((L,), jnp.float32) for _ in range(NCH)))
    inv = tuple(1.0 / s_tot[c] for c in range(NCH))

    def p2_in(j, blk):
        a0 = pl.multiple_of(blk * AB, AB)
        return pltpu.make_async_copy(
            o_hbm.at[pl.ds(a0, AB), pl.ds(bw, BW)], p_v.at[j], p2_sem.at[j])

    def rescale_block(j, blk):
        f = []
        for c in range(NCH):
            st = pl.multiple_of(blk * (2 * BW) + c * L, L)
            f.append(jnp.exp(ms_v[pl.ds(st, L)] - m_fin[c]) * inv[c])

        def pc(a, carry2):
            for c in range(NCH):
                off = c * L
                p_v[j, a, pl.ds(off, L)] = p_v[j, a, pl.ds(off, L)] * f[c]
            return carry2

        lax.fori_loop(0, AB, pc, 0)
        out_slab(j, blk).start()

    for j in range(NBUF):
        p2_in(j, j).start()

    def blk2_body(g, carry):
        for j in range(NBUF):
            blk = g * NBUF + j
            p2_in(j, blk).wait()
            rescale_block(j, blk)

            @pl.when(g < NRING - 1)
            def _():
                out_slab(j, blk).wait()
                p2_in(j, blk + NBUF).start()
        return carry

    lax.fori_loop(0, NRING, blk2_body, 0)
    out_slab(0, NBLK - 3).wait()
    p2_in(0, NBLK - 1).start()
    p2_in(0, NBLK - 1).wait()
    rescale_block(0, NBLK - 1)
    out_slab(1, NBLK - 2).wait()
    out_slab(0, NBLK - 1).wait()


@jax.jit
def _sc_call(at, et):
    return pl.kernel(
        _sc_body,
        out_type=jax.ShapeDtypeStruct((A, B), jnp.float32),
        mesh=plsc.VectorSubcoreMesh(
            core_axis_name="c", subcore_axis_name="s",
            num_cores=NC, num_subcores=NS,
        ),
        scratch_types=[
            pltpu.VMEM((NBUF, E, AB, BW), jnp.float32),
            pltpu.VMEM((NBUF, E, AB, BW), jnp.float32),
            pltpu.VMEM((NBUF, AB, BW), jnp.float32),
            pltpu.VMEM((NBLK * 2 * BW,), jnp.float32),
            pltpu.SemaphoreType.DMA((NBUF,)),
            pltpu.SemaphoreType.DMA((NBUF,)),
            pltpu.SemaphoreType.DMA((NBUF,)),
        ],
        compiler_params=pltpu.CompilerParams(
            use_tc_tiling_on_sc=True, needs_layout_passes=False,
        ),
    )(at, et)


def kernel(action, explore_rate, step, obs):
    del step, obs
    at = jnp.transpose(action, (1, 2, 0))
    et = jnp.transpose(explore_rate, (1, 2, 0))
    out_t = _sc_call(at, et)
    return jnp.transpose(out_t, (1, 0))
